# baseline (speedup 1.0000x reference)
import functools

import jax
import jax.numpy as jnp
from jax import lax
from jax.experimental import pallas as pl
from jax.experimental.pallas import tpu as pltpu
from jax.experimental.pallas import tpu_sc as plsc

NUM_USERS = 1000000
NUM_ITEMS = 1000000
FACTORS = 64
BATCH = 16384

NUM_CORES = 2
NUM_SUBCORES = 16
NW = NUM_CORES * NUM_SUBCORES

BPW = BATCH // NW
IDX_CHUNK = 128
NCHUNK = BPW // IDX_CHUNK


def _gather_u_body(idx_hbm, tab_hbm, rows_out_hbm, idx_v, rows_v, sem, sem_out):
    wid = lax.axis_index("s") * NUM_CORES + lax.axis_index("c")
    base = wid * BPW
    pltpu.sync_copy(idx_hbm.at[pl.ds(wid * NCHUNK, NCHUNK)], idx_v)
    cps = [pltpu.async_copy(tab_hbm.at[idx_v.at[j]],
                            rows_v.at[pl.ds(j * IDX_CHUNK, IDX_CHUNK)], sem)
           for j in range(NCHUNK)]
    for cp in cps:
        cp.wait()
    pltpu.sync_copy(rows_v, rows_out_hbm.at[pl.ds(base, BPW)])


@functools.partial(
    pl.kernel,
    out_type=jax.ShapeDtypeStruct((BATCH, FACTORS), jnp.float32),
    mesh=plsc.VectorSubcoreMesh(core_axis_name="c", subcore_axis_name="s"),
    compiler_params=pltpu.CompilerParams(use_tc_tiling_on_sc=False),
    scratch_types=[
        pltpu.VMEM((NCHUNK, IDX_CHUNK), jnp.int32),
        pltpu.VMEM((BPW, FACTORS), jnp.float32),
        pltpu.SemaphoreType.DMA,
        pltpu.SemaphoreType.DMA,
    ],
)
def _gather_users(idx_hbm, tab_hbm, *rest):
    _gather_u_body(idx_hbm, tab_hbm, *rest)


def _gather_i_body(idx_hbm, tab_hbm, bi_hbm, rows_out_hbm, beta_hbm,
                   idx_v, rows_v, bi_v, sem, sem_b, sem_out):
    wid = lax.axis_index("s") * NUM_CORES + lax.axis_index("c")
    base = wid * BPW
    pltpu.sync_copy(idx_hbm.at[pl.ds(wid * NCHUNK, NCHUNK)], idx_v)
    cps = [pltpu.async_copy(tab_hbm.at[idx_v.at[j]],
                            rows_v.at[pl.ds(j * IDX_CHUNK, IDX_CHUNK)], sem)
           for j in range(NCHUNK)]
    bcs = [pltpu.async_copy(bi_hbm.at[idx_v.at[j]],
                            bi_v.at[pl.ds(j * IDX_CHUNK, IDX_CHUNK)], sem_b)
           for j in range(NCHUNK)]
    for cp in cps:
        cp.wait()
    out_rows = pltpu.async_copy(rows_v, rows_out_hbm.at[pl.ds(base, BPW)],
                                sem_out)
    for cp in bcs:
        cp.wait()
    pltpu.sync_copy(bi_v, beta_hbm.at[pl.ds(base, BPW)])
    out_rows.wait()


@functools.partial(
    pl.kernel,
    out_type=(
        jax.ShapeDtypeStruct((BATCH, FACTORS), jnp.float32),
        jax.ShapeDtypeStruct((BATCH,), jnp.float32),
    ),
    mesh=plsc.VectorSubcoreMesh(core_axis_name="c", subcore_axis_name="s"),
    compiler_params=pltpu.CompilerParams(use_tc_tiling_on_sc=False),
    scratch_types=[
        pltpu.VMEM((NCHUNK, IDX_CHUNK), jnp.int32),
        pltpu.VMEM((BPW, FACTORS), jnp.float32),
        pltpu.VMEM((BPW,), jnp.float32),
        pltpu.SemaphoreType.DMA,
        pltpu.SemaphoreType.DMA,
        pltpu.SemaphoreType.DMA,
    ],
)
def _gather_items(idx_hbm, tab_hbm, bi_hbm, *rest):
    _gather_i_body(idx_hbm, tab_hbm, bi_hbm, *rest)


TC_BLOCK = 2048


def _dot_tc_body(gu_ref, gi_ref, beta_ref, xui_ref):
    xui_ref[...] = beta_ref[...] + jnp.sum(gu_ref[...] * gi_ref[...], axis=1)


_dot_tc = pl.pallas_call(
    _dot_tc_body,
    grid=(BATCH // TC_BLOCK,),
    in_specs=[
        pl.BlockSpec((TC_BLOCK, FACTORS), lambda i: (i, 0)),
        pl.BlockSpec((TC_BLOCK, FACTORS), lambda i: (i, 0)),
        pl.BlockSpec((TC_BLOCK,), lambda i: (i,)),
    ],
    out_specs=pl.BlockSpec((TC_BLOCK,), lambda i: (i,)),
    out_shape=jax.ShapeDtypeStruct((BATCH,), jnp.float32),
)


def kernel(users_indices, items_indices, Gu, Gi, Bi):
    users2d = users_indices.astype(jnp.int32).reshape(NW * NCHUNK, IDX_CHUNK)
    items2d = items_indices.astype(jnp.int32).reshape(NW * NCHUNK, IDX_CHUNK)
    bi_flat = Bi.reshape(NUM_ITEMS)
    gamma_u = _gather_users(users2d, Gu)
    gamma_i, beta_i = _gather_items(items2d, Gi, bi_flat)
    xui = _dot_tc(gamma_u, gamma_i, beta_i)
    return (xui, beta_i, gamma_u, gamma_i)


# speedup vs baseline: 2.5370x; 2.5370x over previous
import functools

import jax
import jax.numpy as jnp
from jax import lax
from jax.experimental import pallas as pl
from jax.experimental.pallas import tpu as pltpu
from jax.experimental.pallas import tpu_sc as plsc

NUM_USERS = 1000000
NUM_ITEMS = 1000000
FACTORS = 64
BATCH = 16384

NUM_CORES = 2
NUM_SUBCORES = 16
NW = NUM_CORES * NUM_SUBCORES
LANES = 16

BPW = BATCH // NW
WIN = 8
NWIN = BPW // WIN
IDX_CHUNK = 128
NCHUNK = BPW // IDX_CHUNK

_MESH = plsc.VectorSubcoreMesh(core_axis_name="c", subcore_axis_name="s")


def _strip_body(sids_hbm, tabT_hbm, outT_hbm, sid_v, strip_v, gt_v, sem):
    wid = lax.axis_index("s") * NUM_CORES + lax.axis_index("c")
    pltpu.sync_copy(sids_hbm.at[pl.ds((wid // 8) * 8, 8)], sid_v)
    row = wid % 8
    fi = lax.iota(jnp.int32, LANES)

    def chunk(k, carry):
        ids16 = sid_v[row, pl.ds(k * LANES, LANES)]
        for half in range(2):
            ids = [ids16[half * WIN + j] for j in range(WIN)]
            cols = [i >> 7 for i in ids]
            offs = [i & 127 for i in ids]
            news, slots = [], []
            nd = jnp.int32(0)
            sprev = jnp.int32(0)
            for j in range(WIN):
                new = (cols[j] != cols[j - 1]) if j else (cols[j] == cols[j])
                slot = jnp.where(new, nd, sprev)
                news.append(new)
                slots.append(slot)
                sprev = slot
                nd = nd + new.astype(jnp.int32)
            descs = [
                pltpu.make_async_copy(
                    tabT_hbm.at[:, pl.ds(cols[j] * 128, 128)],
                    strip_v.at[slots[j]], sem)
                for j in range(WIN)
            ]
            for j in range(WIN):
                @pl.when(news[j])
                def _(d=descs[j]):
                    d.start()
            for j in range(WIN):
                @pl.when(news[j])
                def _(d=descs[j]):
                    d.wait()
            for j in range(WIN):
                rloc = jnp.full((LANES,), k * LANES + half * WIN + j,
                                jnp.int32)
                coff = jnp.full((LANES,), 1, jnp.int32) * offs[j]
                for q in range(FACTORS // LANES):
                    x = plsc.load_gather(strip_v.at[slots[j]],
                                         [fi + q * LANES, coff])
                    plsc.store_scatter(gt_v, [fi + q * LANES, rloc], x)
        return carry

    lax.fori_loop(0, BPW // LANES, chunk, 0)
    pltpu.sync_copy(gt_v, outT_hbm.at[:, pl.ds(wid * BPW, BPW)])


@functools.partial(
    pl.kernel,
    out_type=jax.ShapeDtypeStruct((FACTORS, BATCH), jnp.float32),
    mesh=_MESH,
    compiler_params=pltpu.CompilerParams(
        use_tc_tiling_on_sc=True, needs_layout_passes=False),
    scratch_types=[
        pltpu.VMEM((8, 512), jnp.int32),
        pltpu.VMEM((WIN, FACTORS, 128), jnp.float32),
        pltpu.VMEM((FACTORS, BPW), jnp.float32),
        pltpu.SemaphoreType.DMA,
    ],
)
def _strip_gather(sids_hbm, tabT_hbm, *rest):
    _strip_body(sids_hbm, tabT_hbm, *rest)


def _unsort_body(gsu_hbm, gsi_hbm, pu_hbm, qi_hbm, si_hbm, bi_hbm,
                 gu_hbm, gi_hbm, beta_hbm,
                 pu_v, qi_v, si_v, rows_v, bi_v, sem, semg, semb):
    wid = lax.axis_index("s") * NUM_CORES + lax.axis_index("c")
    base = wid * BPW
    pltpu.sync_copy(pu_hbm.at[pl.ds(wid * NCHUNK, NCHUNK)], pu_v)
    pltpu.sync_copy(qi_hbm.at[pl.ds(wid * NCHUNK, NCHUNK)], qi_v)
    pltpu.sync_copy(si_hbm.at[pl.ds(wid * NCHUNK, NCHUNK)], si_v)
    bcs = [pltpu.async_copy(bi_hbm.at[si_v.at[j]],
                            bi_v.at[pl.ds(j * IDX_CHUNK, IDX_CHUNK)], sem)
           for j in range(NCHUNK)]
    pltpu.sync_copy(gsu_hbm.at[pl.ds(base, BPW)], rows_v)
    ucs = [pltpu.async_copy(rows_v.at[pl.ds(j * IDX_CHUNK, IDX_CHUNK)],
                            gu_hbm.at[pu_v.at[j]], semg)
           for j in range(NCHUNK)]
    for cp in bcs:
        cp.wait()
    bss = [pltpu.async_copy(bi_v.at[pl.ds(j * IDX_CHUNK, IDX_CHUNK)],
                            beta_hbm.at[qi_v.at[j]], semb)
           for j in range(NCHUNK)]
    for cp in ucs:
        cp.wait()
    pltpu.sync_copy(gsi_hbm.at[pl.ds(base, BPW)], rows_v)
    ics = [pltpu.async_copy(rows_v.at[pl.ds(j * IDX_CHUNK, IDX_CHUNK)],
                            gi_hbm.at[qi_v.at[j]], semg)
           for j in range(NCHUNK)]
    for cp in bss:
        cp.wait()
    for cp in ics:
        cp.wait()


@functools.partial(
    pl.kernel,
    out_type=(
        jax.ShapeDtypeStruct((BATCH, FACTORS), jnp.float32),
        jax.ShapeDtypeStruct((BATCH, FACTORS), jnp.float32),
        jax.ShapeDtypeStruct((BATCH,), jnp.float32),
    ),
    mesh=_MESH,
    compiler_params=pltpu.CompilerParams(use_tc_tiling_on_sc=False),
    scratch_types=[
        pltpu.VMEM((NCHUNK, IDX_CHUNK), jnp.int32),
        pltpu.VMEM((NCHUNK, IDX_CHUNK), jnp.int32),
        pltpu.VMEM((NCHUNK, IDX_CHUNK), jnp.int32),
        pltpu.VMEM((BPW, FACTORS), jnp.float32),
        pltpu.VMEM((BPW,), jnp.float32),
        pltpu.SemaphoreType.DMA,
        pltpu.SemaphoreType.DMA,
        pltpu.SemaphoreType.DMA,
    ],
)
def _unsort(gsu_hbm, gsi_hbm, pu_hbm, qi_hbm, si_hbm, bi_hbm, *rest):
    _unsort_body(gsu_hbm, gsi_hbm, pu_hbm, qi_hbm, si_hbm, bi_hbm, *rest)


TC_BLOCK = 2048


def _dot_tc_body(gu_ref, gi_ref, beta_ref, xui_ref):
    xui_ref[...] = beta_ref[...] + jnp.sum(gu_ref[...] * gi_ref[...], axis=1)


_dot_tc = pl.pallas_call(
    _dot_tc_body,
    grid=(BATCH // TC_BLOCK,),
    in_specs=[
        pl.BlockSpec((TC_BLOCK, FACTORS), lambda i: (i, 0)),
        pl.BlockSpec((TC_BLOCK, FACTORS), lambda i: (i, 0)),
        pl.BlockSpec((TC_BLOCK,), lambda i: (i,)),
    ],
    out_specs=pl.BlockSpec((TC_BLOCK,), lambda i: (i,)),
    out_shape=jax.ShapeDtypeStruct((BATCH,), jnp.float32),
)


def kernel(users_indices, items_indices, Gu, Gi, Bi):
    iu = users_indices.astype(jnp.int32)
    ii = items_indices.astype(jnp.int32)
    pos = lax.iota(jnp.int32, BATCH)
    su, pu = lax.sort((iu, pos), num_keys=1)
    si, qi = lax.sort((ii, pos), num_keys=1)
    gsuT = _strip_gather(su.reshape(NW, BPW), Gu.T)
    gsiT = _strip_gather(si.reshape(NW, BPW), Gi.T)
    gamma_u, gamma_i, beta_i = _unsort(
        gsuT.T, gsiT.T,
        pu.reshape(NW * NCHUNK, IDX_CHUNK),
        qi.reshape(NW * NCHUNK, IDX_CHUNK),
        si.reshape(NW * NCHUNK, IDX_CHUNK),
        Bi.reshape(NUM_ITEMS))
    xui = _dot_tc(gamma_u, gamma_i, beta_i)
    return (xui, beta_i, gamma_u, gamma_i)


# speedup vs baseline: 2.6740x; 1.0540x over previous
import functools

import jax
import jax.numpy as jnp
from jax import lax
from jax.experimental import pallas as pl
from jax.experimental.pallas import tpu as pltpu
from jax.experimental.pallas import tpu_sc as plsc

NUM_USERS = 1000000
NUM_ITEMS = 1000000
FACTORS = 64
BATCH = 16384

NUM_CORES = 2
NUM_SUBCORES = 16
NW = NUM_CORES * NUM_SUBCORES
LANES = 16

BPW = BATCH // NW
WIN = 8
NWIN = BPW // WIN
IDX_CHUNK = 128
NCHUNK = BPW // IDX_CHUNK

_MESH = plsc.VectorSubcoreMesh(core_axis_name="c", subcore_axis_name="s")


def _strip_body(sids_hbm, tabT_hbm, outT_hbm, sid_v, strip_v, gt_v, sem):
    wid = lax.axis_index("s") * NUM_CORES + lax.axis_index("c")
    pltpu.sync_copy(sids_hbm.at[pl.ds((wid // 8) * 8, 8)], sid_v)
    row = wid % 8
    fi = lax.iota(jnp.int32, LANES)

    def chunk(k, carry):
        cprev, sprev = carry
        ids16 = sid_v[row, pl.ds(k * LANES, LANES)]
        for half in range(2):
            ids = [ids16[half * WIN + j] for j in range(WIN)]
            cols = [i >> 7 for i in ids]
            offs = [i & 127 for i in ids]
            news, slots = [], []
            nd = jnp.int32(0)
            first_new = cols[0] != cprev
            sbase = jnp.where(first_new, jnp.int32(0), (sprev + 1) % WIN)
            for j in range(WIN):
                new = (cols[j] != cols[j - 1]) if j else first_new
                slot = jnp.where(new, (sbase + nd) % WIN, sprev)
                news.append(new)
                slots.append(slot)
                sprev = slot
                nd = nd + new.astype(jnp.int32)
            cprev = cols[WIN - 1]
            descs = [
                pltpu.make_async_copy(
                    tabT_hbm.at[:, pl.ds(cols[j] * 128, 128)],
                    strip_v.at[slots[j]], sem)
                for j in range(WIN)
            ]
            for j in range(WIN):
                @pl.when(news[j])
                def _(d=descs[j]):
                    d.start()
            for j in range(WIN):
                @pl.when(news[j])
                def _(d=descs[j]):
                    d.wait()
            for j in range(WIN):
                rloc = jnp.full((LANES,), k * LANES + half * WIN + j,
                                jnp.int32)
                coff = jnp.full((LANES,), 1, jnp.int32) * offs[j]
                for q in range(FACTORS // LANES):
                    x = plsc.load_gather(strip_v.at[slots[j]],
                                         [fi + q * LANES, coff])
                    plsc.store_scatter(gt_v, [fi + q * LANES, rloc], x)
        return (cprev, sprev)

    lax.fori_loop(0, BPW // LANES, chunk, (jnp.int32(-1), jnp.int32(0)))
    pltpu.sync_copy(gt_v, outT_hbm.at[:, pl.ds(wid * BPW, BPW)])


@functools.partial(
    pl.kernel,
    out_type=jax.ShapeDtypeStruct((FACTORS, BATCH), jnp.float32),
    mesh=_MESH,
    compiler_params=pltpu.CompilerParams(
        use_tc_tiling_on_sc=True, needs_layout_passes=False),
    scratch_types=[
        pltpu.VMEM((8, 512), jnp.int32),
        pltpu.VMEM((WIN, FACTORS, 128), jnp.float32),
        pltpu.VMEM((FACTORS, BPW), jnp.float32),
        pltpu.SemaphoreType.DMA,
    ],
)
def _strip_gather(sids_hbm, tabT_hbm, *rest):
    _strip_body(sids_hbm, tabT_hbm, *rest)


def _unsort_body(gsu_hbm, gsi_hbm, pu_hbm, qi_hbm, si_hbm, bi_hbm,
                 gu_hbm, gi_hbm, beta_hbm,
                 pu_v, qi_v, si_v, rows_v, bi_v, sem, semg, semb):
    wid = lax.axis_index("s") * NUM_CORES + lax.axis_index("c")
    base = wid * BPW
    pltpu.sync_copy(pu_hbm.at[pl.ds(wid * NCHUNK, NCHUNK)], pu_v)
    pltpu.sync_copy(qi_hbm.at[pl.ds(wid * NCHUNK, NCHUNK)], qi_v)
    pltpu.sync_copy(si_hbm.at[pl.ds(wid * NCHUNK, NCHUNK)], si_v)
    bcs = [pltpu.async_copy(bi_hbm.at[si_v.at[j]],
                            bi_v.at[pl.ds(j * IDX_CHUNK, IDX_CHUNK)], sem)
           for j in range(NCHUNK)]
    pltpu.sync_copy(gsu_hbm.at[pl.ds(base, BPW)], rows_v)
    ucs = [pltpu.async_copy(rows_v.at[pl.ds(j * IDX_CHUNK, IDX_CHUNK)],
                            gu_hbm.at[pu_v.at[j]], semg)
           for j in range(NCHUNK)]
    for cp in bcs:
        cp.wait()
    bss = [pltpu.async_copy(bi_v.at[pl.ds(j * IDX_CHUNK, IDX_CHUNK)],
                            beta_hbm.at[qi_v.at[j]], semb)
           for j in range(NCHUNK)]
    for cp in ucs:
        cp.wait()
    pltpu.sync_copy(gsi_hbm.at[pl.ds(base, BPW)], rows_v)
    ics = [pltpu.async_copy(rows_v.at[pl.ds(j * IDX_CHUNK, IDX_CHUNK)],
                            gi_hbm.at[qi_v.at[j]], semg)
           for j in range(NCHUNK)]
    for cp in bss:
        cp.wait()
    for cp in ics:
        cp.wait()


@functools.partial(
    pl.kernel,
    out_type=(
        jax.ShapeDtypeStruct((BATCH, FACTORS), jnp.float32),
        jax.ShapeDtypeStruct((BATCH, FACTORS), jnp.float32),
        jax.ShapeDtypeStruct((BATCH,), jnp.float32),
    ),
    mesh=_MESH,
    compiler_params=pltpu.CompilerParams(use_tc_tiling_on_sc=False),
    scratch_types=[
        pltpu.VMEM((NCHUNK, IDX_CHUNK), jnp.int32),
        pltpu.VMEM((NCHUNK, IDX_CHUNK), jnp.int32),
        pltpu.VMEM((NCHUNK, IDX_CHUNK), jnp.int32),
        pltpu.VMEM((BPW, FACTORS), jnp.float32),
        pltpu.VMEM((BPW,), jnp.float32),
        pltpu.SemaphoreType.DMA,
        pltpu.SemaphoreType.DMA,
        pltpu.SemaphoreType.DMA,
    ],
)
def _unsort(gsu_hbm, gsi_hbm, pu_hbm, qi_hbm, si_hbm, bi_hbm, *rest):
    _unsort_body(gsu_hbm, gsi_hbm, pu_hbm, qi_hbm, si_hbm, bi_hbm, *rest)


TC_BLOCK = 2048


def _dot_tc_body(gu_ref, gi_ref, beta_ref, xui_ref):
    xui_ref[...] = beta_ref[...] + jnp.sum(gu_ref[...] * gi_ref[...], axis=1)


_dot_tc = pl.pallas_call(
    _dot_tc_body,
    grid=(BATCH // TC_BLOCK,),
    in_specs=[
        pl.BlockSpec((TC_BLOCK, FACTORS), lambda i: (i, 0)),
        pl.BlockSpec((TC_BLOCK, FACTORS), lambda i: (i, 0)),
        pl.BlockSpec((TC_BLOCK,), lambda i: (i,)),
    ],
    out_specs=pl.BlockSpec((TC_BLOCK,), lambda i: (i,)),
    out_shape=jax.ShapeDtypeStruct((BATCH,), jnp.float32),
)


def kernel(users_indices, items_indices, Gu, Gi, Bi):
    iu = users_indices.astype(jnp.int32)
    ii = items_indices.astype(jnp.int32)
    pos = lax.iota(jnp.int32, BATCH)
    su, pu = lax.sort((iu, pos), num_keys=1)
    si, qi = lax.sort((ii, pos), num_keys=1)
    gsuT = _strip_gather(su.reshape(NW, BPW), Gu.T)
    gsiT = _strip_gather(si.reshape(NW, BPW), Gi.T)
    gamma_u, gamma_i, beta_i = _unsort(
        gsuT.T, gsiT.T,
        pu.reshape(NW * NCHUNK, IDX_CHUNK),
        qi.reshape(NW * NCHUNK, IDX_CHUNK),
        si.reshape(NW * NCHUNK, IDX_CHUNK),
        Bi.reshape(NUM_ITEMS))
    xui = _dot_tc(gamma_u, gamma_i, beta_i)
    return (xui, beta_i, gamma_u, gamma_i)
